# baseline (speedup 1.0000x reference)
import functools

import jax
import jax.numpy as jnp
from jax import lax
from jax.experimental import pallas as pl
from jax.experimental.pallas import tpu as pltpu
from jax.experimental.pallas import tpu_sc as plsc

N = 10000
E = 320000
D_IN = 128
DH = 64
G = 64

NC = 2
NS = 16
NW = NC * NS
L = 16

RB = 512
NPAD = 10240
NB = NPAD // RB
SL = NPAD // NS
CH = 128
K = 80
EPAD = NW * K * CH
K0 = 116
K1 = 2 * K - K0
TCHUNKS = NS * (K0 + K1)
SKCH = TCHUNKS + (K0 - K1)

_mesh = plsc.VectorSubcoreMesh(core_axis_name="c", subcore_axis_name="s")
_sc_params = pltpu.CompilerParams(needs_layout_passes=False,
                                  use_tc_tiling_on_sc=False)


@functools.partial(
    pl.kernel,
    out_type=jax.ShapeDtypeStruct((NC, NPAD), jnp.float32),
    mesh=_mesh,
    compiler_params=_sc_params,
    scratch_types=[
        pltpu.VMEM((K * CH,), jnp.int32),
        pltpu.VMEM((NPAD,), jnp.float32),
        pltpu.VMEM((SL,), jnp.float32),
        pltpu.VMEM((SL,), jnp.float32),
        pltpu.VMEM_SHARED((NS, NPAD), jnp.float32),
    ],
)
def _sc_degree(dst_hbm, deg_out, idx_v, hist, acc_v, tmp_v, hist_sh):
    cid = lax.axis_index("c")
    sid = lax.axis_index("s")
    wid = cid * NS + sid
    pltpu.sync_copy(dst_hbm.at[wid], idx_v)

    def zb(i, _):
        hist[pl.ds(i * L, L)] = jnp.zeros((L,), jnp.float32)
        return 0

    lax.fori_loop(0, NPAD // L, zb, 0)

    ones = jnp.ones((L,), jnp.float32)

    def eb(i, _):
        ids = idx_v[pl.ds(i * L, L)]
        plsc.addupdate_scatter(hist, [ids], ones)
        return 0

    lax.fori_loop(0, (K * CH) // L, eb, 0)

    pltpu.sync_copy(hist, hist_sh.at[sid])
    plsc.subcore_barrier()

    def za(i, _):
        acc_v[pl.ds(i * L, L)] = jnp.zeros((L,), jnp.float32)
        return 0

    lax.fori_loop(0, SL // L, za, 0)
    for h in range(NS):
        pltpu.sync_copy(hist_sh.at[h, pl.ds(sid * SL, SL)], tmp_v)

        def ab(i, _):
            sl = pl.ds(i * L, L)
            acc_v[sl] = acc_v[sl] + tmp_v[sl]
            return 0

        lax.fori_loop(0, SL // L, ab, 0)
    pltpu.sync_copy(acc_v, deg_out.at[cid, pl.ds(sid * SL, SL)])


@functools.partial(
    pl.kernel,
    out_type=jax.ShapeDtypeStruct((NC, NPAD, DH), jnp.float32),
    mesh=_mesh,
    compiler_params=_sc_params,
    scratch_types=[
        pltpu.VMEM((K0, CH), jnp.int32),
        pltpu.VMEM((K0, CH), jnp.int32),
        pltpu.VMEM((CH, DH), jnp.float32),
        pltpu.VMEM((CH, DH), jnp.float32),
        pltpu.VMEM_SHARED((NPAD, DH), jnp.float32),
        pltpu.SemaphoreType.DMA,
        pltpu.SemaphoreType.DMA,
    ],
)
def _sc_edge_pass(h_hbm, src_hbm, dst_hbm, zeros_hbm, out_hbm,
                  src_v, dst_v, rows0, rows1, acc_sh, sem0, sem1):
    cid = lax.axis_index("c")
    sid = lax.axis_index("s")
    base = jnp.where(cid == 0, sid * K0, NS * K0 + sid * K1)
    half = jnp.where(cid == 0, K0 // 2, K1 // 2)
    with jax.named_scope("idx_load"):
        pltpu.sync_copy(src_hbm.at[pl.ds(base, K0)], src_v)
        pltpu.sync_copy(dst_hbm.at[pl.ds(base, K0)], dst_v)
    nsl = pl.ds(sid * SL, SL)
    with jax.named_scope("zero_acc"):
        pltpu.sync_copy(zeros_hbm.at[nsl], acc_sh.at[nsl])
    plsc.subcore_barrier()

    pltpu.async_copy(h_hbm.at[src_v.at[0]], rows0, sem0)
    pltpu.async_copy(h_hbm.at[src_v.at[1]], rows1, sem1)

    def body(t, _):
        j0 = 2 * t
        j1 = j0 + 1
        pltpu.make_async_copy(h_hbm.at[src_v.at[j0]], rows0, sem0).wait()
        pltpu.sync_copy(rows0, acc_sh.at[dst_v.at[j0]], add=True)

        @pl.when(t < half - 1)
        def _():
            pltpu.async_copy(h_hbm.at[src_v.at[j0 + 2]], rows0, sem0)

        pltpu.make_async_copy(h_hbm.at[src_v.at[j1]], rows1, sem1).wait()
        pltpu.sync_copy(rows1, acc_sh.at[dst_v.at[j1]], add=True)

        @pl.when(t < half - 1)
        def _():
            pltpu.async_copy(h_hbm.at[src_v.at[j1 + 2]], rows1, sem1)

        return 0

    with jax.named_scope("edge_loop"):
        lax.fori_loop(0, half, body, 0)
    plsc.subcore_barrier()
    with jax.named_scope("writeout"):
        pltpu.sync_copy(acc_sh.at[nsl], out_hbm.at[cid, nsl])


def _tc_a_body(x_ref, w1_ref, d0_ref, d1_ref, h_ref, dis_ref):
    deg = d0_ref[...] + d1_ref[...] + 1.0
    dis = lax.rsqrt(deg)
    h = jnp.dot(x_ref[...], w1_ref[...], preferred_element_type=jnp.float32)
    h_ref[...] = h * dis
    dis_ref[...] = dis


def _tc_a(x_pad, W1, deg0, deg1):
    return pl.pallas_call(
        _tc_a_body,
        grid=(NB,),
        in_specs=[
            pl.BlockSpec((RB, D_IN), lambda i: (i, 0)),
            pl.BlockSpec((D_IN, DH), lambda i: (0, 0)),
            pl.BlockSpec((RB, 1), lambda i: (i, 0)),
            pl.BlockSpec((RB, 1), lambda i: (i, 0)),
        ],
        out_specs=[
            pl.BlockSpec((RB, DH), lambda i: (i, 0)),
            pl.BlockSpec((RB, 1), lambda i: (i, 0)),
        ],
        out_shape=[
            jax.ShapeDtypeStruct((NPAD, DH), jnp.float32),
            jax.ShapeDtypeStruct((NPAD, 1), jnp.float32),
        ],
    )(x_pad, W1, deg0, deg1)


def _tc_b_body(s0_ref, s1_ref, hp_ref, dis_ref, w2_ref, b1_ref, out_ref):
    dis = dis_ref[...]
    z = jax.nn.relu(dis * (s0_ref[...] + s1_ref[...] + hp_ref[...])
                    + b1_ref[...])
    out_ref[...] = jnp.dot(z, w2_ref[...],
                           preferred_element_type=jnp.float32) * dis


def _tc_b(s0, s1, h1p, dis, W2, b1r):
    return pl.pallas_call(
        _tc_b_body,
        grid=(NB,),
        in_specs=[
            pl.BlockSpec((RB, DH), lambda i: (i, 0)),
            pl.BlockSpec((RB, DH), lambda i: (i, 0)),
            pl.BlockSpec((RB, DH), lambda i: (i, 0)),
            pl.BlockSpec((RB, 1), lambda i: (i, 0)),
            pl.BlockSpec((DH, DH), lambda i: (0, 0)),
            pl.BlockSpec((1, DH), lambda i: (0, 0)),
        ],
        out_specs=pl.BlockSpec((RB, DH), lambda i: (i, 0)),
        out_shape=jax.ShapeDtypeStruct((NPAD, DH), jnp.float32),
    )(s0, s1, h1p, dis, W2, b1r)


def _tc_c_body(s0_ref, s1_ref, hp_ref, dis_ref, b2_ref, bt_ref,
               wf1_ref, bf1_ref, wf2_ref, bf2_ref, out_ref,
               sum_acc, max_acc, cnt_acc):
    pid = pl.program_id(0)

    @pl.when(pid == 0)
    def _():
        sum_acc[...] = jnp.zeros((G, DH), jnp.float32)
        max_acc[...] = jnp.full((G, DH), -jnp.inf, jnp.float32)
        cnt_acc[...] = jnp.zeros((G, 1), jnp.float32)

    h2 = (dis_ref[...] * (s0_ref[...] + s1_ref[...] + hp_ref[...])
          + b2_ref[...])
    bt = bt_ref[...]
    row = pid * RB + lax.broadcasted_iota(jnp.int32, (RB, 1), 0)
    valid = row < N
    gi = lax.broadcasted_iota(jnp.int32, (RB, G), 1)
    onehot = jnp.where((bt == gi) & valid, 1.0, 0.0)
    sum_acc[...] += lax.dot_general(
        onehot, h2, (((0,), (0,)), ((), ())),
        preferred_element_type=jnp.float32)
    cnt_acc[...] += lax.dot_general(
        onehot, jnp.ones((RB, 1), jnp.float32), (((0,), (0,)), ((), ())),
        preferred_element_type=jnp.float32)

    h2m = jnp.where(valid, h2, -jnp.inf)
    glo = jnp.min(bt)
    ghi = jnp.max(bt)

    def gbody(g, _):
        m = jnp.where(bt == g, h2m, -jnp.inf)
        colmax = jnp.max(m, axis=0, keepdims=True)
        sl = pl.ds(g, 1)
        max_acc[sl, :] = jnp.maximum(max_acc[sl, :], colmax)
        return 0

    lax.fori_loop(glo, ghi + 1, gbody, 0)

    @pl.when(pid == NB - 1)
    def _():
        mean = sum_acc[...] / jnp.maximum(cnt_acc[...], 1.0)
        pooled = jnp.concatenate([mean, max_acc[...]], axis=1)
        z = jax.nn.relu(
            jnp.dot(pooled, wf1_ref[...], preferred_element_type=jnp.float32)
            + bf1_ref[...])
        o = jnp.dot(z, wf2_ref[...], preferred_element_type=jnp.float32)
        out_ref[...] = o[:, 0:1] + bf2_ref[...]


def _tc_c(s0, s1, h2p, dis, b2r, batch_c, Wfc1, bfc1r, Wfc2p, bfc2r):
    return pl.pallas_call(
        _tc_c_body,
        grid=(NB,),
        in_specs=[
            pl.BlockSpec((RB, DH), lambda i: (i, 0)),
            pl.BlockSpec((RB, DH), lambda i: (i, 0)),
            pl.BlockSpec((RB, DH), lambda i: (i, 0)),
            pl.BlockSpec((RB, 1), lambda i: (i, 0)),
            pl.BlockSpec((1, DH), lambda i: (0, 0)),
            pl.BlockSpec((RB, 1), lambda i: (i, 0)),
            pl.BlockSpec((2 * DH, DH), lambda i: (0, 0)),
            pl.BlockSpec((1, DH), lambda i: (0, 0)),
            pl.BlockSpec((DH, 128), lambda i: (0, 0)),
            pl.BlockSpec((1, 1), lambda i: (0, 0)),
        ],
        out_specs=pl.BlockSpec((G, 1), lambda i: (0, 0)),
        out_shape=jax.ShapeDtypeStruct((G, 1), jnp.float32),
        scratch_shapes=[
            pltpu.VMEM((G, DH), jnp.float32),
            pltpu.VMEM((G, DH), jnp.float32),
            pltpu.VMEM((G, 1), jnp.float32),
        ],
    )(s0, s1, h2p, dis, b2r, batch_c, Wfc1, bfc1r, Wfc2p, bfc2r)


@jax.jit
def kernel(x, edge_index, batch, W1, b1, W2, b2, Wfc1, bfc1, Wfc2, bfc2):
    f32 = jnp.float32
    src = edge_index[0].astype(jnp.int32)
    dst = edge_index[1].astype(jnp.int32)
    pad_s = jnp.full((SKCH * CH - E,), NPAD - 1, jnp.int32)
    src3 = jnp.concatenate([src, pad_s]).reshape(SKCH, CH)
    dst_sk = jnp.concatenate([dst, pad_s])
    dst3 = dst_sk.reshape(SKCH, CH)
    dst2 = dst_sk[:EPAD].reshape(NW, K * CH)

    x_pad = jnp.zeros((NPAD, D_IN), f32).at[:N].set(x)
    batch_c = jnp.full((NPAD, 1), G - 1, jnp.int32).at[:N, 0].set(
        batch.astype(jnp.int32))
    zeros_h = jnp.zeros((NPAD, DH), f32)

    deg_p = _sc_degree(dst2)
    deg0 = deg_p[0].reshape(NPAD, 1)
    deg1 = deg_p[1].reshape(NPAD, 1)

    h1p, dis = _tc_a(x_pad, W1, deg0, deg1)
    s1 = _sc_edge_pass(h1p, src3, dst3, zeros_h)
    h2p = _tc_b(s1[0], s1[1], h1p, dis, W2, b1.reshape(1, DH))
    s2 = _sc_edge_pass(h2p, src3, dst3, zeros_h)

    Wfc2p = jnp.zeros((DH, 128), f32).at[:, 0:1].set(Wfc2)
    out = _tc_c(s2[0], s2[1], h2p, dis, b2.reshape(1, DH), batch_c,
                Wfc1, bfc1.reshape(1, DH), Wfc2p, bfc2.reshape(1, 1))
    return out


# speedup vs baseline: 2.1914x; 2.1914x over previous
import functools

import jax
import jax.numpy as jnp
from jax import lax
from jax.experimental import pallas as pl
from jax.experimental.pallas import tpu as pltpu
from jax.experimental.pallas import tpu_sc as plsc

N = 10000
E = 320000
D_IN = 128
DH = 64
G = 64

NC = 2
NS = 16
NW = NC * NS
L = 16

RB = 512
NPAD = 10240
NB = NPAD // RB
SL = NPAD // NS
CH = 128
NCH = E // CH
KMAX = NCH // NW + 1
XTRA = KMAX * NW - NCH
CPAD = NCH + KMAX

_mesh = plsc.VectorSubcoreMesh(core_axis_name="c", subcore_axis_name="s")
_sc_params = pltpu.CompilerParams(needs_layout_passes=False,
                                  use_tc_tiling_on_sc=False)


@functools.partial(
    pl.kernel,
    out_type=jax.ShapeDtypeStruct((NC, NPAD), jnp.float32),
    mesh=_mesh,
    compiler_params=_sc_params,
    scratch_types=[
        pltpu.VMEM((KMAX * CH,), jnp.int32),
        pltpu.VMEM((NPAD,), jnp.float32),
        pltpu.VMEM((SL,), jnp.float32),
        pltpu.VMEM((SL,), jnp.float32),
        pltpu.VMEM_SHARED((NS, NPAD), jnp.float32),
    ],
)
def _sc_degree(dst_hbm, deg_out, idx_v, hist, acc_v, tmp_v, hist_sh):
    cid = lax.axis_index("c")
    sid = lax.axis_index("s")
    wid = cid * NS + sid
    xt = jnp.minimum(wid, NW - XTRA)
    base = (KMAX - 1) * wid + xt
    cnt = jnp.where(wid < NW - XTRA, KMAX, KMAX - 1)
    pltpu.sync_copy(dst_hbm.at[pl.ds(base * CH, KMAX * CH)], idx_v)

    def zb(i, _):
        hist[pl.ds(i * L, L)] = jnp.zeros((L,), jnp.float32)
        return 0

    lax.fori_loop(0, NPAD // L, zb, 0)

    ones = jnp.ones((L,), jnp.float32)

    def eb(i, _):
        ids = idx_v[pl.ds(i * L, L)]
        plsc.addupdate_scatter(hist, [ids], ones)
        return 0

    lax.fori_loop(0, cnt * (CH // L), eb, 0)

    pltpu.sync_copy(hist, hist_sh.at[sid])
    plsc.subcore_barrier()

    def za(i, _):
        acc_v[pl.ds(i * L, L)] = jnp.zeros((L,), jnp.float32)
        return 0

    lax.fori_loop(0, SL // L, za, 0)
    for h in range(NS):
        pltpu.sync_copy(hist_sh.at[h, pl.ds(sid * SL, SL)], tmp_v)

        def ab(i, _):
            sl = pl.ds(i * L, L)
            acc_v[sl] = acc_v[sl] + tmp_v[sl]
            return 0

        lax.fori_loop(0, SL // L, ab, 0)
    pltpu.sync_copy(acc_v, deg_out.at[cid, pl.ds(sid * SL, SL)])


@functools.partial(
    pl.kernel,
    out_type=jax.ShapeDtypeStruct((NC, NPAD, DH), jnp.float32),
    mesh=_mesh,
    compiler_params=_sc_params,
    scratch_types=[
        pltpu.VMEM((KMAX, CH), jnp.int32),
        pltpu.VMEM((KMAX, CH), jnp.int32),
        pltpu.VMEM((CH, DH), jnp.float32),
        pltpu.VMEM((CH, DH), jnp.float32),
        pltpu.VMEM_SHARED((NPAD, DH), jnp.float32),
        pltpu.SemaphoreType.DMA,
        pltpu.SemaphoreType.DMA,
    ],
)
def _sc_edge_pass(h_hbm, src_hbm, dst_hbm, zeros_hbm, out_hbm,
                  src_v, dst_v, rows0, rows1, acc_sh, sem0, sem1):
    cid = lax.axis_index("c")
    sid = lax.axis_index("s")
    wid = cid * NS + sid
    xt = jnp.minimum(wid, NW - XTRA)
    base = (KMAX - 1) * wid + xt
    cnt = jnp.where(wid < NW - XTRA, KMAX, KMAX - 1)
    with jax.named_scope("idx_load"):
        pltpu.sync_copy(src_hbm.at[pl.ds(base, KMAX)], src_v)
        pltpu.sync_copy(dst_hbm.at[pl.ds(base, KMAX)], dst_v)
    nsl = pl.ds(sid * SL, SL)
    with jax.named_scope("zero_acc"):
        pltpu.sync_copy(zeros_hbm.at[nsl], acc_sh.at[nsl])
    plsc.subcore_barrier()

    pltpu.async_copy(h_hbm.at[src_v.at[0]], rows0, sem0)
    pltpu.async_copy(h_hbm.at[src_v.at[1]], rows1, sem1)

    def body(t, _):
        j0 = 2 * t
        j1 = j0 + 1
        pltpu.make_async_copy(h_hbm.at[src_v.at[j0]], rows0, sem0).wait()
        pltpu.sync_copy(rows0, acc_sh.at[dst_v.at[j0]], add=True)

        @pl.when(j0 + 2 < cnt)
        def _():
            pltpu.async_copy(h_hbm.at[src_v.at[j0 + 2]], rows0, sem0)

        pltpu.make_async_copy(h_hbm.at[src_v.at[j1]], rows1, sem1).wait()
        pltpu.sync_copy(rows1, acc_sh.at[dst_v.at[j1]], add=True)

        @pl.when(j1 + 2 < cnt)
        def _():
            pltpu.async_copy(h_hbm.at[src_v.at[j1 + 2]], rows1, sem1)

        return 0

    with jax.named_scope("edge_loop"):
        lax.fori_loop(0, cnt // 2, body, 0)

        @pl.when(cnt % 2 == 1)
        def _():
            j = cnt - 1
            pltpu.make_async_copy(h_hbm.at[src_v.at[j]], rows0, sem0).wait()
            pltpu.sync_copy(rows0, acc_sh.at[dst_v.at[j]], add=True)

    plsc.subcore_barrier()
    with jax.named_scope("writeout"):
        pltpu.sync_copy(acc_sh.at[nsl], out_hbm.at[cid, nsl])


def _tc_a_body(x_ref, w1_ref, d0_ref, d1_ref, h_ref, dis_ref):
    deg = d0_ref[...] + d1_ref[...] + 1.0
    dis = lax.rsqrt(deg)
    h = jnp.dot(x_ref[...], w1_ref[...], preferred_element_type=jnp.float32)
    h_ref[...] = h * dis
    dis_ref[...] = dis


def _tc_a(x_pad, W1, deg0, deg1):
    return pl.pallas_call(
        _tc_a_body,
        grid=(NB,),
        in_specs=[
            pl.BlockSpec((RB, D_IN), lambda i: (i, 0)),
            pl.BlockSpec((D_IN, DH), lambda i: (0, 0)),
            pl.BlockSpec((RB, 1), lambda i: (i, 0)),
            pl.BlockSpec((RB, 1), lambda i: (i, 0)),
        ],
        out_specs=[
            pl.BlockSpec((RB, DH), lambda i: (i, 0)),
            pl.BlockSpec((RB, 1), lambda i: (i, 0)),
        ],
        out_shape=[
            jax.ShapeDtypeStruct((NPAD, DH), jnp.float32),
            jax.ShapeDtypeStruct((NPAD, 1), jnp.float32),
        ],
    )(x_pad, W1, deg0, deg1)


def _tc_b_body(s0_ref, s1_ref, hp_ref, dis_ref, w2_ref, b1_ref, out_ref):
    dis = dis_ref[...]
    z = jax.nn.relu(dis * (s0_ref[...] + s1_ref[...] + hp_ref[...])
                    + b1_ref[...])
    out_ref[...] = jnp.dot(z, w2_ref[...],
                           preferred_element_type=jnp.float32) * dis


def _tc_b(s0, s1, h1p, dis, W2, b1r):
    return pl.pallas_call(
        _tc_b_body,
        grid=(NB,),
        in_specs=[
            pl.BlockSpec((RB, DH), lambda i: (i, 0)),
            pl.BlockSpec((RB, DH), lambda i: (i, 0)),
            pl.BlockSpec((RB, DH), lambda i: (i, 0)),
            pl.BlockSpec((RB, 1), lambda i: (i, 0)),
            pl.BlockSpec((DH, DH), lambda i: (0, 0)),
            pl.BlockSpec((1, DH), lambda i: (0, 0)),
        ],
        out_specs=pl.BlockSpec((RB, DH), lambda i: (i, 0)),
        out_shape=jax.ShapeDtypeStruct((NPAD, DH), jnp.float32),
    )(s0, s1, h1p, dis, W2, b1r)


def _tc_c_body(s0_ref, s1_ref, hp_ref, dis_ref, b2_ref, bt_ref,
               wf1_ref, bf1_ref, wf2_ref, bf2_ref, out_ref,
               sum_acc, max_acc, cnt_acc):
    pid = pl.program_id(0)

    @pl.when(pid == 0)
    def _():
        sum_acc[...] = jnp.zeros((G, DH), jnp.float32)
        max_acc[...] = jnp.full((G, DH), -jnp.inf, jnp.float32)
        cnt_acc[...] = jnp.zeros((G, 1), jnp.float32)

    h2 = (dis_ref[...] * (s0_ref[...] + s1_ref[...] + hp_ref[...])
          + b2_ref[...])
    bt = bt_ref[...]
    row = pid * RB + lax.broadcasted_iota(jnp.int32, (RB, 1), 0)
    valid = row < N
    gi = lax.broadcasted_iota(jnp.int32, (RB, G), 1)
    onehot = jnp.where((bt == gi) & valid, 1.0, 0.0)
    sum_acc[...] += lax.dot_general(
        onehot, h2, (((0,), (0,)), ((), ())),
        preferred_element_type=jnp.float32)
    cnt_acc[...] += lax.dot_general(
        onehot, jnp.ones((RB, 1), jnp.float32), (((0,), (0,)), ((), ())),
        preferred_element_type=jnp.float32)

    h2m = jnp.where(valid, h2, -jnp.inf)
    glo = jnp.min(bt)
    ghi = jnp.max(bt)

    def gbody(g, _):
        m = jnp.where(bt == g, h2m, -jnp.inf)
        colmax = jnp.max(m, axis=0, keepdims=True)
        sl = pl.ds(g, 1)
        max_acc[sl, :] = jnp.maximum(max_acc[sl, :], colmax)
        return 0

    lax.fori_loop(glo, ghi + 1, gbody, 0)

    @pl.when(pid == NB - 1)
    def _():
        mean = sum_acc[...] / jnp.maximum(cnt_acc[...], 1.0)
        pooled = jnp.concatenate([mean, max_acc[...]], axis=1)
        z = jax.nn.relu(
            jnp.dot(pooled, wf1_ref[...], preferred_element_type=jnp.float32)
            + bf1_ref[...])
        o = jnp.dot(z, wf2_ref[...], preferred_element_type=jnp.float32)
        out_ref[...] = o[:, 0:1] + bf2_ref[...]


def _tc_c(s0, s1, h2p, dis, b2r, batch_c, Wfc1, bfc1r, Wfc2p, bfc2r):
    return pl.pallas_call(
        _tc_c_body,
        grid=(NB,),
        in_specs=[
            pl.BlockSpec((RB, DH), lambda i: (i, 0)),
            pl.BlockSpec((RB, DH), lambda i: (i, 0)),
            pl.BlockSpec((RB, DH), lambda i: (i, 0)),
            pl.BlockSpec((RB, 1), lambda i: (i, 0)),
            pl.BlockSpec((1, DH), lambda i: (0, 0)),
            pl.BlockSpec((RB, 1), lambda i: (i, 0)),
            pl.BlockSpec((2 * DH, DH), lambda i: (0, 0)),
            pl.BlockSpec((1, DH), lambda i: (0, 0)),
            pl.BlockSpec((DH, 128), lambda i: (0, 0)),
            pl.BlockSpec((1, 1), lambda i: (0, 0)),
        ],
        out_specs=pl.BlockSpec((G, 1), lambda i: (0, 0)),
        out_shape=jax.ShapeDtypeStruct((G, 1), jnp.float32),
        scratch_shapes=[
            pltpu.VMEM((G, DH), jnp.float32),
            pltpu.VMEM((G, DH), jnp.float32),
            pltpu.VMEM((G, 1), jnp.float32),
        ],
    )(s0, s1, h2p, dis, b2r, batch_c, Wfc1, bfc1r, Wfc2p, bfc2r)


@jax.jit
def kernel(x, edge_index, batch, W1, b1, W2, b2, Wfc1, bfc1, Wfc2, bfc2):
    f32 = jnp.float32
    src = edge_index[0].astype(jnp.int32)
    dst = edge_index[1].astype(jnp.int32)
    tail = jnp.zeros((KMAX * CH,), jnp.int32)
    src3 = jnp.concatenate([src, tail]).reshape(CPAD, CH)
    dst_p = jnp.concatenate([dst, tail])
    dst3 = dst_p.reshape(CPAD, CH)
    dst2 = dst_p

    x_pad = jnp.zeros((NPAD, D_IN), f32).at[:N].set(x)
    batch_c = jnp.full((NPAD, 1), G - 1, jnp.int32).at[:N, 0].set(
        batch.astype(jnp.int32))
    zeros_h = jnp.zeros((NPAD, DH), f32)

    deg_p = _sc_degree(dst2)
    deg0 = deg_p[0].reshape(NPAD, 1)
    deg1 = deg_p[1].reshape(NPAD, 1)

    h1p, dis = _tc_a(x_pad, W1, deg0, deg1)
    s1 = _sc_edge_pass(h1p, src3, dst3, zeros_h)
    h2p = _tc_b(s1[0], s1[1], h1p, dis, W2, b1.reshape(1, DH))
    s2 = _sc_edge_pass(h2p, src3, dst3, zeros_h)

    Wfc2p = jnp.zeros((DH, 128), f32).at[:, 0:1].set(Wfc2)
    out = _tc_c(s2[0], s2[1], h2p, dis, b2.reshape(1, DH), batch_c,
                Wfc1, bfc1.reshape(1, DH), Wfc2p, bfc2.reshape(1, 1))
    return out
